# baseline (speedup 1.0000x reference)
import jax
import jax.numpy as jnp
from jax.experimental import pallas as pl
from jax.experimental.pallas import tpu as pltpu

_H = 224
_W = 224
_NC = 7
_NPX = _H * _W
_EPS = 1e-7
_TRIMAP_CONF = 100.0


def _i32(x):
    return jnp.asarray(x, jnp.int32)


def _box1(x, axis):
    z = jnp.zeros_like(jax.lax.slice_in_dim(x, 0, 1, axis=axis))
    up = jnp.concatenate([jax.lax.slice_in_dim(x, 1, None, axis=axis), z], axis=axis)
    dn = jnp.concatenate([z, jax.lax.slice_in_dim(x, 0, x.shape[axis] - 1, axis=axis)], axis=axis)
    return up + x + dn


def _box3(x):
    return _box1(_box1(x, x.ndim - 2), x.ndim - 1)


def _loss_kernel(img_ref, p_ref, tri_ref, out_ref, st_ref):
    ci = pl.program_id(0)
    img = img_ref[...]
    p = p_ref[0]
    tri = tri_ref[...]

    f0 = jnp.float32(0.0)
    f1 = jnp.float32(1.0)
    f2 = jnp.float32(2.0)
    f3 = jnp.float32(3.0)
    fconf = jnp.float32(_TRIMAP_CONF)

    ii = jax.lax.broadcasted_iota(jnp.int32, (_H, _W), 0)
    jj = jax.lax.broadcasted_iota(jnp.int32, (_H, _W), 1)
    valid = (ii >= 1) & (ii <= _H - 2) & (jj >= 1) & (jj <= _W - 2)
    vmask = jnp.where(valid, f1, f0)
    rci = jnp.where((ii == 0) | (ii == _H - 1), f1,
                    jnp.where((ii == 1) | (ii == _H - 2), f2, f3))
    rcj = jnp.where((jj == 0) | (jj == _W - 1), f1,
                    jnp.where((jj == 1) | (jj == _W - 2), f2, f3))
    cnt = rci * rcj

    @pl.when(ci == 0)
    def _stats():
        pairs = ((0, 0), (0, 1), (0, 2), (1, 1), (1, 2), (2, 2))
        prods = jnp.stack([img[a] * img[b] for a, b in pairs])
        b1 = _box3(jnp.concatenate([img, prods], axis=0)) * (1.0 / 9.0)
        mu = b1[0:3]
        bp = b1[3:9]
        reg = _EPS / 9.0
        a = bp[0] - mu[0] * mu[0] + reg
        b = bp[1] - mu[0] * mu[1]
        c = bp[2] - mu[0] * mu[2]
        d = bp[3] - mu[1] * mu[1] + reg
        e = bp[4] - mu[1] * mu[2]
        f = bp[5] - mu[2] * mu[2] + reg
        c00 = d * f - e * e
        c01 = c * e - b * f
        c02 = b * e - c * d
        c11 = a * f - c * c
        c12 = b * c - a * e
        c22 = a * d - b * b
        det = a * c00 + b * c01 + c * c02
        rdet = vmask / jnp.where(valid, det, f1)
        inv = jnp.stack([c00 * rdet, c01 * rdet, c02 * rdet,
                         c11 * rdet, c12 * rdet, c22 * rdet])
        st_ref[0:3] = mu
        st_ref[3:9] = inv
        out_ref[...] = jnp.zeros((1, 1), jnp.float32)

    mu0 = st_ref[0]
    mu1 = st_ref[1]
    mu2 = st_ref[2]
    i00 = st_ref[3]
    i01 = st_ref[4]
    i02 = st_ref[5]
    i11 = st_ref[6]
    i12 = st_ref[7]
    i22 = st_ref[8]

    fwd = _box3(jnp.stack([p, p * img[0], p * img[1], p * img[2]]))
    s = fwd[0]
    t0 = fwd[1] - s * mu0
    t1 = fwd[2] - s * mu1
    t2 = fwd[3] - s * mu2
    y0 = i00 * t0 + i01 * t1 + i02 * t2
    y1 = i01 * t0 + i11 * t1 + i12 * t2
    y2 = i02 * t0 + i12 * t1 + i22 * t2
    g = s * vmask - (mu0 * y0 + mu1 * y1 + mu2 * y2)

    Sb = _box3(jnp.stack([g, y0, y1, y2]))
    Ax = cnt * p - (1.0 / 9.0) * (
        Sb[0] + img[0] * Sb[1] + img[1] * Sb[2] + img[2] * Sb[3])

    conf = jnp.where(tri == 128, f0, fconf)
    target = jnp.where(tri == ci + 1, fconf, f0)
    r = Ax + conf * p - target
    total = jnp.sum(r * r) * (1.0 / (float(_NPX) * float(_NPX)))
    out_ref[...] += total[None, None]


def kernel(cprob, img_org, trimap):
    img = (img_org[0].astype(jnp.float32) * (1.0 / 255.0)).transpose(2, 0, 1)
    p = cprob[0].astype(jnp.float32)
    tri = trimap[0].astype(jnp.int32)
    out = pl.pallas_call(
        _loss_kernel,
        grid=(_NC,),
        in_specs=[
            pl.BlockSpec((3, _H, _W), lambda i: (_i32(0), _i32(0), _i32(0))),
            pl.BlockSpec((1, _H, _W), lambda i: (_i32(i), _i32(0), _i32(0))),
            pl.BlockSpec((_H, _W), lambda i: (_i32(0), _i32(0))),
        ],
        out_specs=pl.BlockSpec((1, 1), lambda i: (_i32(0), _i32(0))),
        out_shape=jax.ShapeDtypeStruct((1, 1), jnp.float32),
        scratch_shapes=[pltpu.VMEM((9, _H, _W), jnp.float32)],
    )(img, p, tri)
    return out[0, 0].astype(jnp.float64)


# speedup vs baseline: 1.0633x; 1.0633x over previous
import jax
import jax.numpy as jnp
from jax.experimental import pallas as pl

_H = 224
_W = 224
_NC = 7
_NPX = _H * _W
_EPS = 1e-7
_TRIMAP_CONF = 100.0


def _box1(x, axis):
    z = jnp.zeros_like(jax.lax.slice_in_dim(x, 0, 1, axis=axis))
    up = jnp.concatenate([jax.lax.slice_in_dim(x, 1, None, axis=axis), z], axis=axis)
    dn = jnp.concatenate([z, jax.lax.slice_in_dim(x, 0, x.shape[axis] - 1, axis=axis)], axis=axis)
    return up + x + dn


def _box3(x):
    return _box1(_box1(x, x.ndim - 2), x.ndim - 1)


def _loss_kernel(img_ref, p_ref, tri_ref, out_ref):
    img = img_ref[...]
    p = p_ref[...]
    tri = tri_ref[...]

    f0 = jnp.float32(0.0)
    f1 = jnp.float32(1.0)
    f2 = jnp.float32(2.0)
    f3 = jnp.float32(3.0)
    fconf = jnp.float32(_TRIMAP_CONF)

    ii = jax.lax.broadcasted_iota(jnp.int32, (_H, _W), 0)
    jj = jax.lax.broadcasted_iota(jnp.int32, (_H, _W), 1)
    valid = (ii >= 1) & (ii <= _H - 2) & (jj >= 1) & (jj <= _W - 2)
    vmask = jnp.where(valid, f1, f0)
    rci = jnp.where((ii == 0) | (ii == _H - 1), f1,
                    jnp.where((ii == 1) | (ii == _H - 2), f2, f3))
    rcj = jnp.where((jj == 0) | (jj == _W - 1), f1,
                    jnp.where((jj == 1) | (jj == _W - 2), f2, f3))
    cnt = rci * rcj

    pairs = ((0, 0), (0, 1), (0, 2), (1, 1), (1, 2), (2, 2))
    prods = jnp.stack([img[a] * img[b] for a, b in pairs])
    pimg = (p[:, None, :, :] * img[None, :, :, :]).reshape(3 * _NC, _H, _W)
    stage1 = jnp.concatenate([img, prods, p, pimg], axis=0)
    b1 = _box3(stage1)
    bi = b1[0:3]
    mu = bi * (1.0 / 9.0)
    bp = b1[3:9] * (1.0 / 9.0)
    s = b1[9:9 + _NC]
    u = b1[9 + _NC:].reshape(_NC, 3, _H, _W)

    reg = _EPS / 9.0
    a = bp[0] - mu[0] * mu[0] + reg
    b = bp[1] - mu[0] * mu[1]
    c = bp[2] - mu[0] * mu[2]
    d = bp[3] - mu[1] * mu[1] + reg
    e = bp[4] - mu[1] * mu[2]
    f = bp[5] - mu[2] * mu[2] + reg

    c00 = d * f - e * e
    c01 = c * e - b * f
    c02 = b * e - c * d
    c11 = a * f - c * c
    c12 = b * c - a * e
    c22 = a * d - b * b
    det = a * c00 + b * c01 + c * c02
    rdet = vmask / jnp.where(valid, det, f1)
    i00 = c00 * rdet
    i01 = c01 * rdet
    i02 = c02 * rdet
    i11 = c11 * rdet
    i12 = c12 * rdet
    i22 = c22 * rdet

    t0 = u[:, 0] - s * mu[0]
    t1 = u[:, 1] - s * mu[1]
    t2 = u[:, 2] - s * mu[2]
    y0 = i00 * t0 + i01 * t1 + i02 * t2
    y1 = i01 * t0 + i11 * t1 + i12 * t2
    y2 = i02 * t0 + i12 * t1 + i22 * t2
    g = s * vmask - (mu[0] * y0 + mu[1] * y1 + mu[2] * y2)

    back = jnp.stack([g, y0, y1, y2], axis=1)
    Sb = _box3(back)
    Ax = cnt[None] * p - (1.0 / 9.0) * (
        Sb[:, 0] + img[0] * Sb[:, 1] + img[1] * Sb[:, 2] + img[2] * Sb[:, 3])

    conf = jnp.where(tri == 128, f0, fconf)
    cls = jax.lax.broadcasted_iota(jnp.int32, (_NC, _H, _W), 0) + 1
    target = jnp.where(tri[None] == cls, fconf, f0)
    r = Ax + conf[None] * p - target
    total = jnp.sum(r * r) * (1.0 / (float(_NPX) * float(_NPX)))
    out_ref[...] = total[None, None]


def kernel(cprob, img_org, trimap):
    img = (img_org[0].astype(jnp.float32) * (1.0 / 255.0)).transpose(2, 0, 1)
    p = cprob[0].astype(jnp.float32)
    tri = trimap[0].astype(jnp.int32)
    out = pl.pallas_call(
        _loss_kernel,
        out_shape=jax.ShapeDtypeStruct((1, 1), jnp.float32),
    )(img, p, tri)
    return out[0, 0].astype(jnp.float64)
